# baseline (speedup 1.0000x reference)
import functools

import jax
import jax.numpy as jnp
from jax import lax
from jax.experimental import pallas as pl
from jax.experimental.pallas import tpu as pltpu
from jax.experimental.pallas import tpu_sc as plsc

_N = 10000
_E = 320000
_C = 128
_CH = 128
_NW = 32
_NCHUNK = _E // _CH
_CPW = _NCHUNK // _NW
_CREM = _NCHUNK - _CPW * _NW
_BPC = 13
_BLD = 24
_NPAD = _NCHUNK + 4
_RPS = 640
_RPS_LAST = _N - 15 * _RPS

_BLK = 400
_GRID = _N // _BLK


def _proj_body(xh, xc, xs, w, b, oh, oc, os_):
    for l, (x, o) in enumerate(((xh, oh), (xc, oc), (xs, os_))):
        acc = lax.dot_general(x[...], w[l], (((1,), (1,)), ((), ())),
                              preferred_element_type=jnp.float32)
        acc = acc + b[l][None, :]
        shuf = acc.reshape(_BLK, 4, 2, 16).transpose(0, 1, 3, 2)
        o[...] = shuf.reshape(_BLK, _C).astype(jnp.bfloat16)


def _project(xh, xc, xs, w3, b3):
    bs_x = pl.BlockSpec((_BLK, _C), lambda i: (i, 0))
    return pl.pallas_call(
        _proj_body,
        grid=(_GRID,),
        in_specs=[bs_x, bs_x, bs_x,
                  pl.BlockSpec((3, _C, _C), lambda i: (0, 0, 0)),
                  pl.BlockSpec((3, _C), lambda i: (0, 0))],
        out_specs=[bs_x, bs_x, bs_x],
        out_shape=[jax.ShapeDtypeStruct((_N, _C), jnp.bfloat16)] * 3,
    )(xh, xc, xs, w3, b3)


def _sum_body(p, oh, oc, os_):
    for l, o in enumerate((oh, oc, os_)):
        o[...] = p[l, 0] + p[l, 1]


def _sum_partials(part):
    bs_o = pl.BlockSpec((_BLK, _C), lambda i: (i, 0))
    return pl.pallas_call(
        _sum_body,
        grid=(_GRID,),
        in_specs=[pl.BlockSpec((3, 2, _BLK, _C), lambda i: (0, 0, i, 0))],
        out_specs=[bs_o, bs_o, bs_o],
        out_shape=[jax.ShapeDtypeStruct((_N, _C), jnp.float32)] * 3,
    )(part)


_HI_MASK = -65536


def _mul_rows(rows_i, rowsF, vbuf, vrow, nrows):
    def mul(r16, carry2):
        vv = vbuf[vrow, pl.ds(r16 * 16, 16)]
        for k in range(16):
            v = vv[k]
            r = r16 * 16 + k
            for g in range(4):
                m = rows_i[r, pl.ds(g * 16, 16)]
                a = jax.lax.bitcast_convert_type(m << 16, jnp.float32)
                b = jax.lax.bitcast_convert_type(m & _HI_MASK, jnp.float32)
                rowsF[r, pl.ds(g * 32, 16)] = a * v
                rowsF[r, pl.ds(g * 32 + 16, 16)] = b * v
        return carry2

    lax.fori_loop(0, nrows // 16, mul, None)


def _sc_body(xh, xc, xs, rh, ch, vh, rc, cc, vc, rs, cs, vs, out,
             acc, rowsA, rowsB, rowsF, rbuf, cbuf, vbuf, zbuf,
             semA, semB, semI, semZ):
    c = lax.axis_index("c")
    s = lax.axis_index("s")
    w = c * 16 + s
    start_w = _CPW * w + jnp.minimum(w, _CREM)
    rbase = s * _RPS

    z16 = jnp.zeros((16,), jnp.float32)

    def zb_body(r, carry):
        for j in range(8):
            zbuf[r, pl.ds(j * 16, 16)] = z16
        return carry

    lax.fori_loop(0, 64, zb_body, None)

    for l, (x_ref, r_ref, c_ref, v_ref) in enumerate(
            ((xh, rh, ch, vh), (xc, rc, cc, vc), (xs, rs, cs, vs))):
        @pl.when(s < 15)
        def _zero_full():
            for k in range(_RPS // 64):
                pltpu.async_copy(zbuf, acc.at[pl.ds(rbase + k * 64, 64)], semZ)
            for k in range(_RPS // 64):
                pltpu.make_async_copy(
                    zbuf, acc.at[pl.ds(rbase + k * 64, 64)], semZ).wait()

        @pl.when(s == 15)
        def _zero_last():
            for k in range(_RPS_LAST // 64):
                pltpu.async_copy(zbuf, acc.at[pl.ds(rbase + k * 64, 64)], semZ)
            rem = _RPS_LAST - (_RPS_LAST // 64) * 64
            if rem:
                pltpu.async_copy(zbuf.at[pl.ds(0, rem)],
                                 acc.at[pl.ds(rbase + _RPS_LAST - rem, rem)],
                                 semZ)
            for k in range(_RPS_LAST // 64):
                pltpu.make_async_copy(
                    zbuf, acc.at[pl.ds(rbase + k * 64, 64)], semZ).wait()
            if rem:
                pltpu.make_async_copy(
                    zbuf.at[pl.ds(0, rem)],
                    acc.at[pl.ds(rbase + _RPS_LAST - rem, rem)], semZ).wait()

        plsc.subcore_barrier()

        def block(b, carry):
            base = start_w + _BPC * b
            a0 = pl.multiple_of((base // 8) * 8, 8)
            o = base - a0
            pltpu.async_copy(r_ref.at[pl.ds(a0, _BLD)], rbuf, semI)
            pltpu.async_copy(c_ref.at[pl.ds(a0, _BLD)], cbuf, semI)
            pltpu.async_copy(v_ref.at[pl.ds(a0, _BLD)], vbuf, semI)
            pltpu.make_async_copy(r_ref.at[pl.ds(a0, _BLD)], rbuf, semI).wait()
            pltpu.make_async_copy(c_ref.at[pl.ds(a0, _BLD)], cbuf, semI).wait()
            pltpu.make_async_copy(v_ref.at[pl.ds(a0, _BLD)], vbuf, semI).wait()

            pltpu.async_copy(x_ref.at[cbuf.at[o]], rowsA, semA)

            def chunk(j, carry2):
                even = (j % 2) == 0

                @pl.when(jnp.logical_and(even, j < _BPC - 1))
                def _pfB():
                    pltpu.async_copy(x_ref.at[cbuf.at[o + j + 1]], rowsB, semB)

                @pl.when(jnp.logical_and(~even, j < _BPC - 1))
                def _pfA():
                    pltpu.async_copy(x_ref.at[cbuf.at[o + j + 1]], rowsA, semA)

                @pl.when(even)
                def _procA():
                    pltpu.make_async_copy(x_ref.at[pl.ds(0, _CH)], rowsA,
                                          semA).wait()
                    _mul_rows(rowsA, rowsF, vbuf, o + j, _CH)
                    pltpu.sync_copy(rowsF, acc.at[rbuf.at[o + j]], add=True)

                @pl.when(~even)
                def _procB():
                    pltpu.make_async_copy(x_ref.at[pl.ds(0, _CH)], rowsB,
                                          semB).wait()
                    _mul_rows(rowsB, rowsF, vbuf, o + j, _CH)
                    pltpu.sync_copy(rowsF, acc.at[rbuf.at[o + j]], add=True)

                return carry2

            lax.fori_loop(0, _BPC, chunk, None)
            return carry

        lax.fori_loop(0, _CPW // _BPC, block, None)

        @pl.when(w < _CREM)
        def _extra():
            base = start_w + _CPW
            a0 = pl.multiple_of((base // 8) * 8, 8)
            o = base - a0
            pltpu.sync_copy(r_ref.at[pl.ds(a0, 8)], rbuf.at[pl.ds(0, 8)])
            pltpu.sync_copy(c_ref.at[pl.ds(a0, 8)], cbuf.at[pl.ds(0, 8)])
            pltpu.sync_copy(v_ref.at[pl.ds(a0, 8)], vbuf.at[pl.ds(0, 8)])
            pltpu.async_copy(x_ref.at[cbuf.at[o]], rowsA, semA).wait()
            _mul_rows(rowsA, rowsF, vbuf, o, _CH)
            pltpu.sync_copy(rowsF, acc.at[rbuf.at[o]], add=True)

        plsc.subcore_barrier()

        @pl.when(s < 15)
        def _flush_full():
            pltpu.sync_copy(acc.at[pl.ds(rbase, _RPS)],
                            out.at[l, c, pl.ds(rbase, _RPS)])

        @pl.when(s == 15)
        def _flush_last():
            pltpu.sync_copy(acc.at[pl.ds(rbase, _RPS_LAST)],
                            out.at[l, c, pl.ds(rbase, _RPS_LAST)])


def _sc_spmm(xh, xc, xs, rh, ch, vh, rc, cc, vc, rs, cs, vs):
    mesh = plsc.VectorSubcoreMesh(core_axis_name="c", subcore_axis_name="s")
    f = pl.kernel(
        _sc_body,
        mesh=mesh,
        compiler_params=pltpu.CompilerParams(use_tc_tiling_on_sc=False),
        out_type=jax.ShapeDtypeStruct((3, 2, _N, _C), jnp.float32),
        scratch_types=[
            pltpu.VMEM_SHARED((_N, _C), jnp.float32),
            pltpu.VMEM((_CH, _C // 2), jnp.int32),
            pltpu.VMEM((_CH, _C // 2), jnp.int32),
            pltpu.VMEM((_CH, _C), jnp.float32),
            pltpu.VMEM((_BLD, _CH), jnp.int32),
            pltpu.VMEM((_BLD, _CH), jnp.int32),
            pltpu.VMEM((_BLD, _CH), jnp.float32),
            pltpu.VMEM((64, _C), jnp.float32),
            pltpu.SemaphoreType.DMA,
            pltpu.SemaphoreType.DMA,
            pltpu.SemaphoreType.DMA,
            pltpu.SemaphoreType.DMA,
        ],
    )
    return f(xh, xc, xs, rh, ch, vh, rc, cc, vc, rs, cs, vs)


def _chunk_grid(idx, val):
    rows = jnp.pad(idx[0].astype(jnp.int32).reshape(_NCHUNK, _CH),
                   ((0, _NPAD - _NCHUNK), (0, 0)))
    cols = jnp.pad(idx[1].astype(jnp.int32).reshape(_NCHUNK, _CH),
                   ((0, _NPAD - _NCHUNK), (0, 0)))
    vals = jnp.pad(val.reshape(_NCHUNK, _CH), ((0, _NPAD - _NCHUNK), (0, 0)))
    return rows, cols, vals


def kernel(X_HypGNet, X_CGNet, X_SGNet, L_hyp_idx, L_hyp_val, L_cg_idx,
           L_cg_val, L_sg_idx, L_sg_val, W_hyp, b_hyp, W_cg, b_cg, W_sg, b_sg):
    w3 = jnp.stack([W_hyp, W_cg, W_sg])
    b3 = jnp.stack([b_hyp, b_cg, b_sg])
    xh, xc, xs = _project(X_HypGNet, X_CGNet, X_SGNet, w3, b3)
    xh, xc, xs = (
        jax.lax.bitcast_convert_type(x.reshape(_N, _C // 2, 2),
                                     jnp.int32)
        for x in (xh, xc, xs))
    rh, ch, vh = _chunk_grid(L_hyp_idx, L_hyp_val)
    rc, cc, vc = _chunk_grid(L_cg_idx, L_cg_val)
    rs, cs, vs = _chunk_grid(L_sg_idx, L_sg_val)
    part = _sc_spmm(xh, xc, xs, rh, ch, vh, rc, cc, vc, rs, cs, vs)
    oh, oc, os_ = _sum_partials(part)
    return (oh, oc, os_)


# speedup vs baseline: 2.9130x; 2.9130x over previous
import functools

import jax
import jax.numpy as jnp
from jax import lax
from jax.experimental import pallas as pl
from jax.experimental.pallas import tpu as pltpu
from jax.experimental.pallas import tpu_sc as plsc

_N = 10000
_E = 320000
_C = 128
_CH = 128
_NW = 32
_NCHUNK = _E // _CH
_CPW = _NCHUNK // _NW
_CREM = _NCHUNK - _CPW * _NW
_BPC = 13
_BLD = 24
_NPAD = _NCHUNK + 4
_RPS = 640
_RPS_LAST = _N - 15 * _RPS

_BLK = 400
_GRID = _N // _BLK


def _proj_body(xh, xc, xs, w, b, oh, oc, os_):
    for l, (x, o) in enumerate(((xh, oh), (xc, oc), (xs, os_))):
        acc = lax.dot_general(x[...], w[l], (((1,), (1,)), ((), ())),
                              preferred_element_type=jnp.float32)
        o[...] = acc + b[l][None, :]


def _project(xh, xc, xs, w3, b3):
    bs_x = pl.BlockSpec((_BLK, _C), lambda i: (i, 0))
    return pl.pallas_call(
        _proj_body,
        grid=(_GRID,),
        in_specs=[bs_x, bs_x, bs_x,
                  pl.BlockSpec((3, _C, _C), lambda i: (0, 0, 0)),
                  pl.BlockSpec((3, _C), lambda i: (0, 0))],
        out_specs=[bs_x, bs_x, bs_x],
        out_shape=[jax.ShapeDtypeStruct((_N, _C), jnp.float32)] * 3,
    )(xh, xc, xs, w3, b3)


def _sum_body(p, oh, oc, os_):
    for l, o in enumerate((oh, oc, os_)):
        o[...] = p[l, 0] + p[l, 1]


def _sum_partials(part):
    bs_o = pl.BlockSpec((_BLK, _C), lambda i: (i, 0))
    return pl.pallas_call(
        _sum_body,
        grid=(_GRID,),
        in_specs=[pl.BlockSpec((3, 2, _BLK, _C), lambda i: (0, 0, i, 0))],
        out_specs=[bs_o, bs_o, bs_o],
        out_shape=[jax.ShapeDtypeStruct((_N, _C), jnp.float32)] * 3,
    )(part)


def _mul_rows(rows_ref, vbuf, vrow, nrows):
    def mul(r16, carry2):
        vv = vbuf[vrow, pl.ds(r16 * 16, 16)]
        for k in range(16):
            v = vv[k]
            r = r16 * 16 + k
            for j in range(8):
                sl = pl.ds(j * 16, 16)
                rows_ref[r, sl] = rows_ref[r, sl] * v
        return carry2

    lax.fori_loop(0, nrows // 16, mul, None)


def _sc_body(xh, xc, xs, rh, ch, vh, rc, cc, vc, rs, cs, vs, out,
             acc, rowsA, rowsB, rbuf, cbuf, vbuf, zbuf,
             semA, semB, semI, semZ, semSA, semSB):
    c = lax.axis_index("c")
    s = lax.axis_index("s")
    w = c * 16 + s
    start_w = _CPW * w + jnp.minimum(w, _CREM)
    rbase = s * _RPS

    z16 = jnp.zeros((16,), jnp.float32)

    def zb_body(r, carry):
        for j in range(8):
            zbuf[r, pl.ds(j * 16, 16)] = z16
        return carry

    lax.fori_loop(0, 64, zb_body, None)

    for l, (x_ref, r_ref, c_ref, v_ref) in enumerate(
            ((xh, rh, ch, vh), (xc, rc, cc, vc), (xs, rs, cs, vs))):
        @pl.when(s < 15)
        def _zero_full():
            for k in range(_RPS // 64):
                pltpu.async_copy(zbuf, acc.at[pl.ds(rbase + k * 64, 64)], semZ)
            for k in range(_RPS // 64):
                pltpu.make_async_copy(
                    zbuf, acc.at[pl.ds(rbase + k * 64, 64)], semZ).wait()

        @pl.when(s == 15)
        def _zero_last():
            for k in range(_RPS_LAST // 64):
                pltpu.async_copy(zbuf, acc.at[pl.ds(rbase + k * 64, 64)], semZ)
            rem = _RPS_LAST - (_RPS_LAST // 64) * 64
            if rem:
                pltpu.async_copy(zbuf.at[pl.ds(0, rem)],
                                 acc.at[pl.ds(rbase + _RPS_LAST - rem, rem)],
                                 semZ)
            for k in range(_RPS_LAST // 64):
                pltpu.make_async_copy(
                    zbuf, acc.at[pl.ds(rbase + k * 64, 64)], semZ).wait()
            if rem:
                pltpu.make_async_copy(
                    zbuf.at[pl.ds(0, rem)],
                    acc.at[pl.ds(rbase + _RPS_LAST - rem, rem)], semZ).wait()

        plsc.subcore_barrier()

        def block(b, carry):
            base = start_w + _BPC * b
            a0 = pl.multiple_of((base // 8) * 8, 8)
            o = base - a0
            pltpu.async_copy(r_ref.at[pl.ds(a0, _BLD)], rbuf, semI)
            pltpu.async_copy(c_ref.at[pl.ds(a0, _BLD)], cbuf, semI)
            pltpu.async_copy(v_ref.at[pl.ds(a0, _BLD)], vbuf, semI)
            pltpu.make_async_copy(r_ref.at[pl.ds(a0, _BLD)], rbuf, semI).wait()
            pltpu.make_async_copy(c_ref.at[pl.ds(a0, _BLD)], cbuf, semI).wait()
            pltpu.make_async_copy(v_ref.at[pl.ds(a0, _BLD)], vbuf, semI).wait()

            pltpu.async_copy(x_ref.at[cbuf.at[o]], rowsA, semA)

            def chunk(j, carry2):
                even = (j % 2) == 0

                @pl.when(jnp.logical_and(even, j >= 2))
                def _wsB():
                    pltpu.make_async_copy(rowsB, acc.at[rbuf.at[o]],
                                          semSB).wait()

                @pl.when(jnp.logical_and(even, j < _BPC - 1))
                def _pfB():
                    pltpu.async_copy(x_ref.at[cbuf.at[o + j + 1]], rowsB, semB)

                @pl.when(~even)
                def _wsA():
                    pltpu.make_async_copy(rowsA, acc.at[rbuf.at[o]],
                                          semSA).wait()

                @pl.when(jnp.logical_and(~even, j < _BPC - 1))
                def _pfA():
                    pltpu.async_copy(x_ref.at[cbuf.at[o + j + 1]], rowsA, semA)

                @pl.when(even)
                def _procA():
                    pltpu.make_async_copy(x_ref.at[pl.ds(0, _CH)], rowsA,
                                          semA).wait()
                    _mul_rows(rowsA, vbuf, o + j, _CH)
                    pltpu.async_copy(rowsA, acc.at[rbuf.at[o + j]], semSA,
                                     add=True)

                @pl.when(~even)
                def _procB():
                    pltpu.make_async_copy(x_ref.at[pl.ds(0, _CH)], rowsB,
                                          semB).wait()
                    _mul_rows(rowsB, vbuf, o + j, _CH)
                    pltpu.async_copy(rowsB, acc.at[rbuf.at[o + j]], semSB,
                                     add=True)

                return carry2

            lax.fori_loop(0, _BPC, chunk, None)
            pltpu.make_async_copy(rowsA, acc.at[rbuf.at[o]], semSA).wait()
            return carry

        lax.fori_loop(0, _CPW // _BPC, block, None)

        @pl.when(w < _CREM)
        def _extra():
            base = start_w + _CPW
            a0 = pl.multiple_of((base // 8) * 8, 8)
            o = base - a0
            pltpu.sync_copy(r_ref.at[pl.ds(a0, 8)], rbuf.at[pl.ds(0, 8)])
            pltpu.sync_copy(c_ref.at[pl.ds(a0, 8)], cbuf.at[pl.ds(0, 8)])
            pltpu.sync_copy(v_ref.at[pl.ds(a0, 8)], vbuf.at[pl.ds(0, 8)])
            pltpu.async_copy(x_ref.at[cbuf.at[o]], rowsA, semA).wait()
            _mul_rows(rowsA, vbuf, o, _CH)
            pltpu.sync_copy(rowsA, acc.at[rbuf.at[o]], add=True)

        plsc.subcore_barrier()

        @pl.when(s < 15)
        def _flush_full():
            pltpu.sync_copy(acc.at[pl.ds(rbase, _RPS)],
                            out.at[l, c, pl.ds(rbase, _RPS)])

        @pl.when(s == 15)
        def _flush_last():
            pltpu.sync_copy(acc.at[pl.ds(rbase, _RPS_LAST)],
                            out.at[l, c, pl.ds(rbase, _RPS_LAST)])


def _sc_spmm(xh, xc, xs, rh, ch, vh, rc, cc, vc, rs, cs, vs):
    mesh = plsc.VectorSubcoreMesh(core_axis_name="c", subcore_axis_name="s")
    f = pl.kernel(
        _sc_body,
        mesh=mesh,
        out_type=jax.ShapeDtypeStruct((3, 2, _N, _C), jnp.float32),
        scratch_types=[
            pltpu.VMEM_SHARED((_N, _C), jnp.float32),
            pltpu.VMEM((_CH, _C), jnp.float32),
            pltpu.VMEM((_CH, _C), jnp.float32),
            pltpu.VMEM((_BLD, _CH), jnp.int32),
            pltpu.VMEM((_BLD, _CH), jnp.int32),
            pltpu.VMEM((_BLD, _CH), jnp.float32),
            pltpu.VMEM((64, _C), jnp.float32),
            pltpu.SemaphoreType.DMA,
            pltpu.SemaphoreType.DMA,
            pltpu.SemaphoreType.DMA,
            pltpu.SemaphoreType.DMA,
            pltpu.SemaphoreType.DMA,
            pltpu.SemaphoreType.DMA,
        ],
    )
    return f(xh, xc, xs, rh, ch, vh, rc, cc, vc, rs, cs, vs)


def _chunk_grid(idx, val):
    rows = jnp.pad(idx[0].astype(jnp.int32).reshape(_NCHUNK, _CH),
                   ((0, _NPAD - _NCHUNK), (0, 0)))
    cols = jnp.pad(idx[1].astype(jnp.int32).reshape(_NCHUNK, _CH),
                   ((0, _NPAD - _NCHUNK), (0, 0)))
    vals = jnp.pad(val.reshape(_NCHUNK, _CH), ((0, _NPAD - _NCHUNK), (0, 0)))
    return rows, cols, vals


def kernel(X_HypGNet, X_CGNet, X_SGNet, L_hyp_idx, L_hyp_val, L_cg_idx,
           L_cg_val, L_sg_idx, L_sg_val, W_hyp, b_hyp, W_cg, b_cg, W_sg, b_sg):
    w3 = jnp.stack([W_hyp, W_cg, W_sg])
    b3 = jnp.stack([b_hyp, b_cg, b_sg])
    xh, xc, xs = _project(X_HypGNet, X_CGNet, X_SGNet, w3, b3)
    rh, ch, vh = _chunk_grid(L_hyp_idx, L_hyp_val)
    rc, cc, vc = _chunk_grid(L_cg_idx, L_cg_val)
    rs, cs, vs = _chunk_grid(L_sg_idx, L_sg_val)
    part = _sc_spmm(xh, xc, xs, rh, ch, vh, rc, cc, vc, rs, cs, vs)
    oh, oc, os_ = _sum_partials(part)
    return (oh, oc, os_)


# speedup vs baseline: 2.9189x; 1.0020x over previous
import functools

import jax
import jax.numpy as jnp
from jax import lax
from jax.experimental import pallas as pl
from jax.experimental.pallas import tpu as pltpu
from jax.experimental.pallas import tpu_sc as plsc

_N = 10000
_E = 320000
_C = 128
_CH = 128
_NW = 32
_NCHUNK = _E // _CH
_CPW = _NCHUNK // _NW
_CREM = _NCHUNK - _CPW * _NW
_BPC = 13
_BLD = 24
_NPAD = _NCHUNK + 4
_RPS = 640
_RPS_LAST = _N - 15 * _RPS

_BLK = 400
_GRID = _N // _BLK


def _proj_body(xh, xc, xs, w, b, oh, oc, os_):
    for l, (x, o) in enumerate(((xh, oh), (xc, oc), (xs, os_))):
        acc = lax.dot_general(x[...], w[l], (((1,), (1,)), ((), ())),
                              preferred_element_type=jnp.float32)
        o[...] = acc + b[l][None, :]


def _project(xh, xc, xs, w3, b3):
    bs_x = pl.BlockSpec((_BLK, _C), lambda i: (i, 0))
    return pl.pallas_call(
        _proj_body,
        grid=(_GRID,),
        in_specs=[bs_x, bs_x, bs_x,
                  pl.BlockSpec((3, _C, _C), lambda i: (0, 0, 0)),
                  pl.BlockSpec((3, _C), lambda i: (0, 0))],
        out_specs=[bs_x, bs_x, bs_x],
        out_shape=[jax.ShapeDtypeStruct((_N, _C), jnp.float32)] * 3,
    )(xh, xc, xs, w3, b3)


def _sum_body(p, oh, oc, os_):
    for l, o in enumerate((oh, oc, os_)):
        o[...] = p[l, 0] + p[l, 1]


def _sum_partials(part):
    bs_o = pl.BlockSpec((_BLK, _C), lambda i: (i, 0))
    return pl.pallas_call(
        _sum_body,
        grid=(_GRID,),
        in_specs=[pl.BlockSpec((3, 2, _BLK, _C), lambda i: (0, 0, i, 0))],
        out_specs=[bs_o, bs_o, bs_o],
        out_shape=[jax.ShapeDtypeStruct((_N, _C), jnp.float32)] * 3,
    )(part)


def _mul_rows(rows_ref, vbuf, vrow, nrows):
    @plsc.parallel_loop(0, nrows // 16, 1, unroll=2)
    def mul(r16):
        vv = vbuf[vrow, pl.ds(r16 * 16, 16)]
        for k in range(16):
            v = vv[k]
            r = r16 * 16 + k
            for j in range(8):
                sl = pl.ds(j * 16, 16)
                rows_ref[r, sl] = rows_ref[r, sl] * v


def _sc_body(xh, xc, xs, rh, ch, vh, rc, cc, vc, rs, cs, vs, out,
             acc, rowsA, rowsB, rbuf, cbuf, vbuf, zbuf,
             semA, semB, semI, semZ, semSA, semSB):
    c = lax.axis_index("c")
    s = lax.axis_index("s")
    w = c * 16 + s
    start_w = _CPW * w + jnp.minimum(w, _CREM)
    rbase = s * _RPS

    z16 = jnp.zeros((16,), jnp.float32)

    def zb_body(r, carry):
        for j in range(8):
            zbuf[r, pl.ds(j * 16, 16)] = z16
        return carry

    lax.fori_loop(0, 64, zb_body, None)

    for l, (x_ref, r_ref, c_ref, v_ref) in enumerate(
            ((xh, rh, ch, vh), (xc, rc, cc, vc), (xs, rs, cs, vs))):
        @pl.when(s < 15)
        def _zero_full():
            for k in range(_RPS // 64):
                pltpu.async_copy(zbuf, acc.at[pl.ds(rbase + k * 64, 64)], semZ)
            for k in range(_RPS // 64):
                pltpu.make_async_copy(
                    zbuf, acc.at[pl.ds(rbase + k * 64, 64)], semZ).wait()

        @pl.when(s == 15)
        def _zero_last():
            for k in range(_RPS_LAST // 64):
                pltpu.async_copy(zbuf, acc.at[pl.ds(rbase + k * 64, 64)], semZ)
            rem = _RPS_LAST - (_RPS_LAST // 64) * 64
            if rem:
                pltpu.async_copy(zbuf.at[pl.ds(0, rem)],
                                 acc.at[pl.ds(rbase + _RPS_LAST - rem, rem)],
                                 semZ)
            for k in range(_RPS_LAST // 64):
                pltpu.make_async_copy(
                    zbuf, acc.at[pl.ds(rbase + k * 64, 64)], semZ).wait()
            if rem:
                pltpu.make_async_copy(
                    zbuf.at[pl.ds(0, rem)],
                    acc.at[pl.ds(rbase + _RPS_LAST - rem, rem)], semZ).wait()

        plsc.subcore_barrier()

        def block(b, carry):
            base = start_w + _BPC * b
            a0 = pl.multiple_of((base // 8) * 8, 8)
            o = base - a0
            pltpu.async_copy(r_ref.at[pl.ds(a0, _BLD)], rbuf, semI)
            pltpu.async_copy(c_ref.at[pl.ds(a0, _BLD)], cbuf, semI)
            pltpu.async_copy(v_ref.at[pl.ds(a0, _BLD)], vbuf, semI)
            pltpu.make_async_copy(r_ref.at[pl.ds(a0, _BLD)], rbuf, semI).wait()
            pltpu.make_async_copy(c_ref.at[pl.ds(a0, _BLD)], cbuf, semI).wait()
            pltpu.make_async_copy(v_ref.at[pl.ds(a0, _BLD)], vbuf, semI).wait()

            pltpu.async_copy(x_ref.at[cbuf.at[o]], rowsA, semA)

            def chunk(j, carry2):
                even = (j % 2) == 0

                @pl.when(jnp.logical_and(even, j >= 2))
                def _wsB():
                    pltpu.make_async_copy(rowsB, acc.at[rbuf.at[o]],
                                          semSB).wait()

                @pl.when(jnp.logical_and(even, j < _BPC - 1))
                def _pfB():
                    pltpu.async_copy(x_ref.at[cbuf.at[o + j + 1]], rowsB, semB)

                @pl.when(~even)
                def _wsA():
                    pltpu.make_async_copy(rowsA, acc.at[rbuf.at[o]],
                                          semSA).wait()

                @pl.when(jnp.logical_and(~even, j < _BPC - 1))
                def _pfA():
                    pltpu.async_copy(x_ref.at[cbuf.at[o + j + 1]], rowsA, semA)

                @pl.when(even)
                def _procA():
                    pltpu.make_async_copy(x_ref.at[pl.ds(0, _CH)], rowsA,
                                          semA).wait()
                    _mul_rows(rowsA, vbuf, o + j, _CH)
                    pltpu.async_copy(rowsA, acc.at[rbuf.at[o + j]], semSA,
                                     add=True)

                @pl.when(~even)
                def _procB():
                    pltpu.make_async_copy(x_ref.at[pl.ds(0, _CH)], rowsB,
                                          semB).wait()
                    _mul_rows(rowsB, vbuf, o + j, _CH)
                    pltpu.async_copy(rowsB, acc.at[rbuf.at[o + j]], semSB,
                                     add=True)

                return carry2

            lax.fori_loop(0, _BPC, chunk, None)
            pltpu.make_async_copy(rowsA, acc.at[rbuf.at[o]], semSA).wait()
            return carry

        lax.fori_loop(0, _CPW // _BPC, block, None)

        @pl.when(w < _CREM)
        def _extra():
            base = start_w + _CPW
            a0 = pl.multiple_of((base // 8) * 8, 8)
            o = base - a0
            pltpu.sync_copy(r_ref.at[pl.ds(a0, 8)], rbuf.at[pl.ds(0, 8)])
            pltpu.sync_copy(c_ref.at[pl.ds(a0, 8)], cbuf.at[pl.ds(0, 8)])
            pltpu.sync_copy(v_ref.at[pl.ds(a0, 8)], vbuf.at[pl.ds(0, 8)])
            pltpu.async_copy(x_ref.at[cbuf.at[o]], rowsA, semA).wait()
            _mul_rows(rowsA, vbuf, o, _CH)
            pltpu.sync_copy(rowsA, acc.at[rbuf.at[o]], add=True)

        plsc.subcore_barrier()

        @pl.when(s < 15)
        def _flush_full():
            pltpu.sync_copy(acc.at[pl.ds(rbase, _RPS)],
                            out.at[l, c, pl.ds(rbase, _RPS)])

        @pl.when(s == 15)
        def _flush_last():
            pltpu.sync_copy(acc.at[pl.ds(rbase, _RPS_LAST)],
                            out.at[l, c, pl.ds(rbase, _RPS_LAST)])


def _sc_spmm(xh, xc, xs, rh, ch, vh, rc, cc, vc, rs, cs, vs):
    mesh = plsc.VectorSubcoreMesh(core_axis_name="c", subcore_axis_name="s")
    f = pl.kernel(
        _sc_body,
        mesh=mesh,
        out_type=jax.ShapeDtypeStruct((3, 2, _N, _C), jnp.float32),
        scratch_types=[
            pltpu.VMEM_SHARED((_N, _C), jnp.float32),
            pltpu.VMEM((_CH, _C), jnp.float32),
            pltpu.VMEM((_CH, _C), jnp.float32),
            pltpu.VMEM((_BLD, _CH), jnp.int32),
            pltpu.VMEM((_BLD, _CH), jnp.int32),
            pltpu.VMEM((_BLD, _CH), jnp.float32),
            pltpu.VMEM((64, _C), jnp.float32),
            pltpu.SemaphoreType.DMA,
            pltpu.SemaphoreType.DMA,
            pltpu.SemaphoreType.DMA,
            pltpu.SemaphoreType.DMA,
            pltpu.SemaphoreType.DMA,
            pltpu.SemaphoreType.DMA,
        ],
    )
    return f(xh, xc, xs, rh, ch, vh, rc, cc, vc, rs, cs, vs)


def _chunk_grid(idx, val):
    rows = jnp.pad(idx[0].astype(jnp.int32).reshape(_NCHUNK, _CH),
                   ((0, _NPAD - _NCHUNK), (0, 0)))
    cols = jnp.pad(idx[1].astype(jnp.int32).reshape(_NCHUNK, _CH),
                   ((0, _NPAD - _NCHUNK), (0, 0)))
    vals = jnp.pad(val.reshape(_NCHUNK, _CH), ((0, _NPAD - _NCHUNK), (0, 0)))
    return rows, cols, vals


def kernel(X_HypGNet, X_CGNet, X_SGNet, L_hyp_idx, L_hyp_val, L_cg_idx,
           L_cg_val, L_sg_idx, L_sg_val, W_hyp, b_hyp, W_cg, b_cg, W_sg, b_sg):
    w3 = jnp.stack([W_hyp, W_cg, W_sg])
    b3 = jnp.stack([b_hyp, b_cg, b_sg])
    xh, xc, xs = _project(X_HypGNet, X_CGNet, X_SGNet, w3, b3)
    rh, ch, vh = _chunk_grid(L_hyp_idx, L_hyp_val)
    rc, cc, vc = _chunk_grid(L_cg_idx, L_cg_val)
    rs, cs, vs = _chunk_grid(L_sg_idx, L_sg_val)
    part = _sc_spmm(xh, xc, xs, rh, ch, vh, rc, cc, vc, rs, cs, vs)
    oh, oc, os_ = _sum_partials(part)
    return (oh, oc, os_)


# speedup vs baseline: 2.9209x; 1.0007x over previous
import functools

import jax
import jax.numpy as jnp
from jax import lax
from jax.experimental import pallas as pl
from jax.experimental.pallas import tpu as pltpu
from jax.experimental.pallas import tpu_sc as plsc

_N = 10000
_E = 320000
_C = 128
_CH = 128
_NW = 32
_NCHUNK = _E // _CH
_CPW = _NCHUNK // _NW
_CREM = _NCHUNK - _CPW * _NW
_BPC = 13
_BLD = 24
_NPAD = _NCHUNK + 4
_RPS = 640
_RPS_LAST = _N - 15 * _RPS

_BLK = 400
_GRID = _N // _BLK


def _proj_body(xh, xc, xs, w, b, oh, oc, os_):
    for l, (x, o) in enumerate(((xh, oh), (xc, oc), (xs, os_))):
        acc = lax.dot_general(x[...], w[l], (((1,), (1,)), ((), ())),
                              preferred_element_type=jnp.float32)
        o[...] = acc + b[l][None, :]


def _project(xh, xc, xs, w3, b3):
    bs_x = pl.BlockSpec((_BLK, _C), lambda i: (i, 0))
    return pl.pallas_call(
        _proj_body,
        grid=(_GRID,),
        in_specs=[bs_x, bs_x, bs_x,
                  pl.BlockSpec((3, _C, _C), lambda i: (0, 0, 0)),
                  pl.BlockSpec((3, _C), lambda i: (0, 0))],
        out_specs=[bs_x, bs_x, bs_x],
        out_shape=[jax.ShapeDtypeStruct((_N, _C), jnp.float32)] * 3,
    )(xh, xc, xs, w3, b3)


def _sum_body(p, oh, oc, os_):
    for l, o in enumerate((oh, oc, os_)):
        o[...] = p[l, 0] + p[l, 1]


def _sum_partials(part):
    bs_o = pl.BlockSpec((_BLK, _C), lambda i: (i, 0))
    return pl.pallas_call(
        _sum_body,
        grid=(_GRID,),
        in_specs=[pl.BlockSpec((3, 2, _BLK, _C), lambda i: (0, 0, i, 0))],
        out_specs=[bs_o, bs_o, bs_o],
        out_shape=[jax.ShapeDtypeStruct((_N, _C), jnp.float32)] * 3,
    )(part)


def _mul_rows(rows_ref, vbuf, vrow, nrows):
    @plsc.parallel_loop(0, nrows // 16, 1, unroll=2)
    def mul(r16):
        vv = vbuf[vrow, pl.ds(r16 * 16, 16)]
        for k in range(16):
            v = vv[k]
            r = r16 * 16 + k
            for j in range(8):
                sl = pl.ds(j * 16, 16)
                rows_ref[r, sl] = rows_ref[r, sl] * v


def _sc_body(xh, xc, xs, rh, ch, vh, rc, cc, vc, rs, cs, vs, out,
             acc, rowsA, rowsB, rbuf, cbuf, vbuf, zbuf,
             semA, semB, semI, semZ, semSA, semSB):
    c = lax.axis_index("c")
    s = lax.axis_index("s")
    w = c * 16 + s
    start_w = _CPW * w + jnp.minimum(w, _CREM)
    rbase = s * _RPS

    z16 = jnp.zeros((16,), jnp.float32)

    def zb_body(r, carry):
        for j in range(8):
            zbuf[r, pl.ds(j * 16, 16)] = z16
        return carry

    lax.fori_loop(0, 64, zb_body, None)

    for l, (x_ref, r_ref, c_ref, v_ref) in enumerate(
            ((xh, rh, ch, vh), (xc, rc, cc, vc), (xs, rs, cs, vs))):
        @pl.when(s < 15)
        def _zero_full():
            for k in range(_RPS // 64):
                pltpu.async_copy(zbuf, acc.at[pl.ds(rbase + k * 64, 64)], semZ)
            for k in range(_RPS // 64):
                pltpu.make_async_copy(
                    zbuf, acc.at[pl.ds(rbase + k * 64, 64)], semZ).wait()

        @pl.when(s == 15)
        def _zero_last():
            for k in range(_RPS_LAST // 64):
                pltpu.async_copy(zbuf, acc.at[pl.ds(rbase + k * 64, 64)], semZ)
            rem = _RPS_LAST - (_RPS_LAST // 64) * 64
            if rem:
                pltpu.async_copy(zbuf.at[pl.ds(0, rem)],
                                 acc.at[pl.ds(rbase + _RPS_LAST - rem, rem)],
                                 semZ)
            for k in range(_RPS_LAST // 64):
                pltpu.make_async_copy(
                    zbuf, acc.at[pl.ds(rbase + k * 64, 64)], semZ).wait()
            if rem:
                pltpu.make_async_copy(
                    zbuf.at[pl.ds(0, rem)],
                    acc.at[pl.ds(rbase + _RPS_LAST - rem, rem)], semZ).wait()

        plsc.subcore_barrier()

        def block(b, carry):
            base = start_w + _BPC * b
            a0 = pl.multiple_of((base // 8) * 8, 8)
            o = base - a0
            pltpu.async_copy(r_ref.at[pl.ds(a0, _BLD)], rbuf, semI)
            pltpu.async_copy(c_ref.at[pl.ds(a0, _BLD)], cbuf, semI)
            pltpu.async_copy(v_ref.at[pl.ds(a0, _BLD)], vbuf, semI)
            pltpu.make_async_copy(r_ref.at[pl.ds(a0, _BLD)], rbuf, semI).wait()
            pltpu.make_async_copy(c_ref.at[pl.ds(a0, _BLD)], cbuf, semI).wait()
            pltpu.make_async_copy(v_ref.at[pl.ds(a0, _BLD)], vbuf, semI).wait()

            pltpu.async_copy(x_ref.at[cbuf.at[o]], rowsA, semA)

            def chunk(j, carry2):
                even = (j % 2) == 0

                @pl.when(even)
                def _procA():
                    pltpu.make_async_copy(x_ref.at[pl.ds(0, _CH)], rowsA,
                                          semA).wait()

                    @pl.when(j >= 2)
                    def _wsB():
                        pltpu.make_async_copy(rowsB, acc.at[rbuf.at[o]],
                                              semSB).wait()

                    @pl.when(j < _BPC - 1)
                    def _pfB():
                        pltpu.async_copy(x_ref.at[cbuf.at[o + j + 1]], rowsB,
                                         semB)

                    _mul_rows(rowsA, vbuf, o + j, _CH)
                    pltpu.async_copy(rowsA, acc.at[rbuf.at[o + j]], semSA,
                                     add=True)

                @pl.when(~even)
                def _procB():
                    pltpu.make_async_copy(x_ref.at[pl.ds(0, _CH)], rowsB,
                                          semB).wait()

                    pltpu.make_async_copy(rowsA, acc.at[rbuf.at[o]],
                                          semSA).wait()

                    @pl.when(j < _BPC - 1)
                    def _pfA():
                        pltpu.async_copy(x_ref.at[cbuf.at[o + j + 1]], rowsA,
                                         semA)

                    _mul_rows(rowsB, vbuf, o + j, _CH)
                    pltpu.async_copy(rowsB, acc.at[rbuf.at[o + j]], semSB,
                                     add=True)

                return carry2

            lax.fori_loop(0, _BPC, chunk, None)
            pltpu.make_async_copy(rowsA, acc.at[rbuf.at[o]], semSA).wait()
            return carry

        lax.fori_loop(0, _CPW // _BPC, block, None)

        @pl.when(w < _CREM)
        def _extra():
            base = start_w + _CPW
            a0 = pl.multiple_of((base // 8) * 8, 8)
            o = base - a0
            pltpu.sync_copy(r_ref.at[pl.ds(a0, 8)], rbuf.at[pl.ds(0, 8)])
            pltpu.sync_copy(c_ref.at[pl.ds(a0, 8)], cbuf.at[pl.ds(0, 8)])
            pltpu.sync_copy(v_ref.at[pl.ds(a0, 8)], vbuf.at[pl.ds(0, 8)])
            pltpu.async_copy(x_ref.at[cbuf.at[o]], rowsA, semA).wait()
            _mul_rows(rowsA, vbuf, o, _CH)
            pltpu.sync_copy(rowsA, acc.at[rbuf.at[o]], add=True)

        plsc.subcore_barrier()

        @pl.when(s < 15)
        def _flush_full():
            pltpu.sync_copy(acc.at[pl.ds(rbase, _RPS)],
                            out.at[l, c, pl.ds(rbase, _RPS)])

        @pl.when(s == 15)
        def _flush_last():
            pltpu.sync_copy(acc.at[pl.ds(rbase, _RPS_LAST)],
                            out.at[l, c, pl.ds(rbase, _RPS_LAST)])


def _sc_spmm(xh, xc, xs, rh, ch, vh, rc, cc, vc, rs, cs, vs):
    mesh = plsc.VectorSubcoreMesh(core_axis_name="c", subcore_axis_name="s")
    f = pl.kernel(
        _sc_body,
        mesh=mesh,
        out_type=jax.ShapeDtypeStruct((3, 2, _N, _C), jnp.float32),
        scratch_types=[
            pltpu.VMEM_SHARED((_N, _C), jnp.float32),
            pltpu.VMEM((_CH, _C), jnp.float32),
            pltpu.VMEM((_CH, _C), jnp.float32),
            pltpu.VMEM((_BLD, _CH), jnp.int32),
            pltpu.VMEM((_BLD, _CH), jnp.int32),
            pltpu.VMEM((_BLD, _CH), jnp.float32),
            pltpu.VMEM((64, _C), jnp.float32),
            pltpu.SemaphoreType.DMA,
            pltpu.SemaphoreType.DMA,
            pltpu.SemaphoreType.DMA,
            pltpu.SemaphoreType.DMA,
            pltpu.SemaphoreType.DMA,
            pltpu.SemaphoreType.DMA,
        ],
    )
    return f(xh, xc, xs, rh, ch, vh, rc, cc, vc, rs, cs, vs)


def _chunk_grid(idx, val):
    rows = jnp.pad(idx[0].astype(jnp.int32).reshape(_NCHUNK, _CH),
                   ((0, _NPAD - _NCHUNK), (0, 0)))
    cols = jnp.pad(idx[1].astype(jnp.int32).reshape(_NCHUNK, _CH),
                   ((0, _NPAD - _NCHUNK), (0, 0)))
    vals = jnp.pad(val.reshape(_NCHUNK, _CH), ((0, _NPAD - _NCHUNK), (0, 0)))
    return rows, cols, vals


def kernel(X_HypGNet, X_CGNet, X_SGNet, L_hyp_idx, L_hyp_val, L_cg_idx,
           L_cg_val, L_sg_idx, L_sg_val, W_hyp, b_hyp, W_cg, b_cg, W_sg, b_sg):
    w3 = jnp.stack([W_hyp, W_cg, W_sg])
    b3 = jnp.stack([b_hyp, b_cg, b_sg])
    xh, xc, xs = _project(X_HypGNet, X_CGNet, X_SGNet, w3, b3)
    rh, ch, vh = _chunk_grid(L_hyp_idx, L_hyp_val)
    rc, cc, vc = _chunk_grid(L_cg_idx, L_cg_val)
    rs, cs, vs = _chunk_grid(L_sg_idx, L_sg_val)
    part = _sc_spmm(xh, xc, xs, rh, ch, vh, rc, cc, vc, rs, cs, vs)
    oh, oc, os_ = _sum_partials(part)
    return (oh, oc, os_)


# speedup vs baseline: 2.9351x; 1.0049x over previous
import functools

import jax
import jax.numpy as jnp
from jax import lax
from jax.experimental import pallas as pl
from jax.experimental.pallas import tpu as pltpu
from jax.experimental.pallas import tpu_sc as plsc

_N = 10000
_E = 320000
_C = 128
_CH = 128
_NW = 32
_NCHUNK = _E // _CH
_CPW = _NCHUNK // _NW
_CREM = _NCHUNK - _CPW * _NW
_BPC = 13
_BLD = 24
_NPAD = _NCHUNK + 4
_RPS = 640
_RPS_LAST = _N - 15 * _RPS

_BLK = 400
_GRID = _N // _BLK


def _proj_body(xh, xc, xs, w, b, oh, oc, os_):
    for l, (x, o) in enumerate(((xh, oh), (xc, oc), (xs, os_))):
        acc = lax.dot_general(x[...], w[l], (((1,), (1,)), ((), ())),
                              preferred_element_type=jnp.float32)
        o[...] = acc + b[l][None, :]


def _project(xh, xc, xs, w3, b3):
    bs_x = pl.BlockSpec((_BLK, _C), lambda i: (i, 0))
    return pl.pallas_call(
        _proj_body,
        grid=(_GRID,),
        in_specs=[bs_x, bs_x, bs_x,
                  pl.BlockSpec((3, _C, _C), lambda i: (0, 0, 0)),
                  pl.BlockSpec((3, _C), lambda i: (0, 0))],
        out_specs=[bs_x, bs_x, bs_x],
        out_shape=[jax.ShapeDtypeStruct((_N, _C), jnp.float32)] * 3,
    )(xh, xc, xs, w3, b3)


def _sum_body(p, oh, oc, os_):
    for l, o in enumerate((oh, oc, os_)):
        o[...] = p[l, 0] + p[l, 1]


def _sum_partials(part):
    bs_o = pl.BlockSpec((_BLK, _C), lambda i: (i, 0))
    return pl.pallas_call(
        _sum_body,
        grid=(_GRID,),
        in_specs=[pl.BlockSpec((3, 2, _BLK, _C), lambda i: (0, 0, i, 0))],
        out_specs=[bs_o, bs_o, bs_o],
        out_shape=[jax.ShapeDtypeStruct((_N, _C), jnp.float32)] * 3,
    )(part)


def _mul_rows(rows_ref, vbuf, vrow, nrows):
    @plsc.parallel_loop(0, nrows // 16, 1, unroll=2)
    def mul(r16):
        vv = vbuf[vrow, pl.ds(r16 * 16, 16)]
        for k in range(16):
            v = vv[k]
            r = r16 * 16 + k
            for j in range(8):
                sl = pl.ds(j * 16, 16)
                rows_ref[r, sl] = rows_ref[r, sl] * v


def _sc_body(xh, xc, xs, rh, ch, vh, rc, cc, vc, rs, cs, vs, out,
             acc, rowsA, rowsB, rbuf, cbuf, vbuf, zbuf,
             semA, semB, semI, semZ, semSA, semSB, semR, semV):
    c = lax.axis_index("c")
    s = lax.axis_index("s")
    w = c * 16 + s
    start_w = _CPW * w + jnp.minimum(w, _CREM)
    rbase = s * _RPS

    z16 = jnp.zeros((16,), jnp.float32)

    def zb_body(r, carry):
        for j in range(8):
            zbuf[r, pl.ds(j * 16, 16)] = z16
        return carry

    lax.fori_loop(0, 64, zb_body, None)

    for l, (x_ref, r_ref, c_ref, v_ref) in enumerate(
            ((xh, rh, ch, vh), (xc, rc, cc, vc), (xs, rs, cs, vs))):
        @pl.when(s < 15)
        def _zero_full():
            for k in range(_RPS // 64):
                pltpu.async_copy(zbuf, acc.at[pl.ds(rbase + k * 64, 64)], semZ)
            for k in range(_RPS // 64):
                pltpu.make_async_copy(
                    zbuf, acc.at[pl.ds(rbase + k * 64, 64)], semZ).wait()

        @pl.when(s == 15)
        def _zero_last():
            for k in range(_RPS_LAST // 64):
                pltpu.async_copy(zbuf, acc.at[pl.ds(rbase + k * 64, 64)], semZ)
            rem = _RPS_LAST - (_RPS_LAST // 64) * 64
            if rem:
                pltpu.async_copy(zbuf.at[pl.ds(0, rem)],
                                 acc.at[pl.ds(rbase + _RPS_LAST - rem, rem)],
                                 semZ)
            for k in range(_RPS_LAST // 64):
                pltpu.make_async_copy(
                    zbuf, acc.at[pl.ds(rbase + k * 64, 64)], semZ).wait()
            if rem:
                pltpu.make_async_copy(
                    zbuf.at[pl.ds(0, rem)],
                    acc.at[pl.ds(rbase + _RPS_LAST - rem, rem)], semZ).wait()

        plsc.subcore_barrier()

        def block(b, carry):
            base = start_w + _BPC * b
            a0 = pl.multiple_of((base // 8) * 8, 8)
            o = base - a0
            pltpu.async_copy(c_ref.at[pl.ds(a0, _BLD)], cbuf, semI)
            pltpu.async_copy(r_ref.at[pl.ds(a0, _BLD)], rbuf, semR)
            pltpu.async_copy(v_ref.at[pl.ds(a0, _BLD)], vbuf, semV)
            pltpu.make_async_copy(c_ref.at[pl.ds(a0, _BLD)], cbuf, semI).wait()

            pltpu.async_copy(x_ref.at[cbuf.at[o]], rowsA, semA)
            pltpu.make_async_copy(r_ref.at[pl.ds(a0, _BLD)], rbuf, semR).wait()
            pltpu.make_async_copy(v_ref.at[pl.ds(a0, _BLD)], vbuf, semV).wait()

            def chunk(j, carry2):
                even = (j % 2) == 0

                @pl.when(even)
                def _procA():
                    pltpu.make_async_copy(x_ref.at[pl.ds(0, _CH)], rowsA,
                                          semA).wait()

                    @pl.when(j >= 2)
                    def _wsB():
                        pltpu.make_async_copy(rowsB, acc.at[rbuf.at[o]],
                                              semSB).wait()

                    @pl.when(j < _BPC - 1)
                    def _pfB():
                        pltpu.async_copy(x_ref.at[cbuf.at[o + j + 1]], rowsB,
                                         semB)

                    _mul_rows(rowsA, vbuf, o + j, _CH)
                    pltpu.async_copy(rowsA, acc.at[rbuf.at[o + j]], semSA,
                                     add=True)

                @pl.when(~even)
                def _procB():
                    pltpu.make_async_copy(x_ref.at[pl.ds(0, _CH)], rowsB,
                                          semB).wait()

                    pltpu.make_async_copy(rowsA, acc.at[rbuf.at[o]],
                                          semSA).wait()

                    @pl.when(j < _BPC - 1)
                    def _pfA():
                        pltpu.async_copy(x_ref.at[cbuf.at[o + j + 1]], rowsA,
                                         semA)

                    _mul_rows(rowsB, vbuf, o + j, _CH)
                    pltpu.async_copy(rowsB, acc.at[rbuf.at[o + j]], semSB,
                                     add=True)

                return carry2

            lax.fori_loop(0, _BPC, chunk, None)
            pltpu.make_async_copy(rowsA, acc.at[rbuf.at[o]], semSA).wait()
            return carry

        lax.fori_loop(0, _CPW // _BPC, block, None)

        @pl.when(w < _CREM)
        def _extra():
            base = start_w + _CPW
            a0 = pl.multiple_of((base // 8) * 8, 8)
            o = base - a0
            pltpu.sync_copy(r_ref.at[pl.ds(a0, 8)], rbuf.at[pl.ds(0, 8)])
            pltpu.sync_copy(c_ref.at[pl.ds(a0, 8)], cbuf.at[pl.ds(0, 8)])
            pltpu.sync_copy(v_ref.at[pl.ds(a0, 8)], vbuf.at[pl.ds(0, 8)])
            pltpu.async_copy(x_ref.at[cbuf.at[o]], rowsA, semA).wait()
            _mul_rows(rowsA, vbuf, o, _CH)
            pltpu.sync_copy(rowsA, acc.at[rbuf.at[o]], add=True)

        plsc.subcore_barrier()

        @pl.when(s < 15)
        def _flush_full():
            pltpu.sync_copy(acc.at[pl.ds(rbase, _RPS)],
                            out.at[l, c, pl.ds(rbase, _RPS)])

        @pl.when(s == 15)
        def _flush_last():
            pltpu.sync_copy(acc.at[pl.ds(rbase, _RPS_LAST)],
                            out.at[l, c, pl.ds(rbase, _RPS_LAST)])


def _sc_spmm(xh, xc, xs, rh, ch, vh, rc, cc, vc, rs, cs, vs):
    mesh = plsc.VectorSubcoreMesh(core_axis_name="c", subcore_axis_name="s")
    f = pl.kernel(
        _sc_body,
        mesh=mesh,
        out_type=jax.ShapeDtypeStruct((3, 2, _N, _C), jnp.float32),
        scratch_types=[
            pltpu.VMEM_SHARED((_N, _C), jnp.float32),
            pltpu.VMEM((_CH, _C), jnp.float32),
            pltpu.VMEM((_CH, _C), jnp.float32),
            pltpu.VMEM((_BLD, _CH), jnp.int32),
            pltpu.VMEM((_BLD, _CH), jnp.int32),
            pltpu.VMEM((_BLD, _CH), jnp.float32),
            pltpu.VMEM((64, _C), jnp.float32),
            pltpu.SemaphoreType.DMA,
            pltpu.SemaphoreType.DMA,
            pltpu.SemaphoreType.DMA,
            pltpu.SemaphoreType.DMA,
            pltpu.SemaphoreType.DMA,
            pltpu.SemaphoreType.DMA,
            pltpu.SemaphoreType.DMA,
            pltpu.SemaphoreType.DMA,
        ],
    )
    return f(xh, xc, xs, rh, ch, vh, rc, cc, vc, rs, cs, vs)


def _chunk_grid(idx, val):
    rows = jnp.pad(idx[0].astype(jnp.int32).reshape(_NCHUNK, _CH),
                   ((0, _NPAD - _NCHUNK), (0, 0)))
    cols = jnp.pad(idx[1].astype(jnp.int32).reshape(_NCHUNK, _CH),
                   ((0, _NPAD - _NCHUNK), (0, 0)))
    vals = jnp.pad(val.reshape(_NCHUNK, _CH), ((0, _NPAD - _NCHUNK), (0, 0)))
    return rows, cols, vals


def kernel(X_HypGNet, X_CGNet, X_SGNet, L_hyp_idx, L_hyp_val, L_cg_idx,
           L_cg_val, L_sg_idx, L_sg_val, W_hyp, b_hyp, W_cg, b_cg, W_sg, b_sg):
    w3 = jnp.stack([W_hyp, W_cg, W_sg])
    b3 = jnp.stack([b_hyp, b_cg, b_sg])
    xh, xc, xs = _project(X_HypGNet, X_CGNet, X_SGNet, w3, b3)
    rh, ch, vh = _chunk_grid(L_hyp_idx, L_hyp_val)
    rc, cc, vc = _chunk_grid(L_cg_idx, L_cg_val)
    rs, cs, vs = _chunk_grid(L_sg_idx, L_sg_val)
    part = _sc_spmm(xh, xc, xs, rh, ch, vh, rc, cc, vc, rs, cs, vs)
    oh, oc, os_ = _sum_partials(part)
    return (oh, oc, os_)


# speedup vs baseline: 2.9366x; 1.0005x over previous
import functools

import jax
import jax.numpy as jnp
from jax import lax
from jax.experimental import pallas as pl
from jax.experimental.pallas import tpu as pltpu
from jax.experimental.pallas import tpu_sc as plsc

_N = 10000
_E = 320000
_C = 128
_CH = 128
_NW = 32
_NCHUNK = _E // _CH
_CPW = _NCHUNK // _NW
_CREM = _NCHUNK - _CPW * _NW
_BPC = 13
_BLD = 24
_NPAD = _NCHUNK + 4
_RPS = 640
_RPS_LAST = _N - 15 * _RPS

_BLK = 400
_GRID = _N // _BLK


def _proj_body(xh, xc, xs, w, b, oh, oc, os_):
    for l, (x, o) in enumerate(((xh, oh), (xc, oc), (xs, os_))):
        acc = lax.dot_general(x[...], w[l], (((1,), (1,)), ((), ())),
                              preferred_element_type=jnp.float32)
        o[...] = acc + b[l][None, :]


def _project(xh, xc, xs, w3, b3):
    bs_x = pl.BlockSpec((_BLK, _C), lambda i: (i, 0))
    return pl.pallas_call(
        _proj_body,
        grid=(_GRID,),
        in_specs=[bs_x, bs_x, bs_x,
                  pl.BlockSpec((3, _C, _C), lambda i: (0, 0, 0)),
                  pl.BlockSpec((3, _C), lambda i: (0, 0))],
        out_specs=[bs_x, bs_x, bs_x],
        out_shape=[jax.ShapeDtypeStruct((_N, _C), jnp.float32)] * 3,
    )(xh, xc, xs, w3, b3)


def _sum_body(p, oh, oc, os_):
    for l, o in enumerate((oh, oc, os_)):
        o[...] = p[l, 0] + p[l, 1]


def _sum_partials(part):
    bs_o = pl.BlockSpec((_BLK, _C), lambda i: (i, 0))
    return pl.pallas_call(
        _sum_body,
        grid=(_GRID,),
        in_specs=[pl.BlockSpec((3, 2, _BLK, _C), lambda i: (0, 0, i, 0))],
        out_specs=[bs_o, bs_o, bs_o],
        out_shape=[jax.ShapeDtypeStruct((_N, _C), jnp.float32)] * 3,
    )(part)


def _mul_rows(rows_ref, vbuf, vrow, nrows):
    @plsc.parallel_loop(0, nrows // 16, 1, unroll=4)
    def mul(r16):
        vv = vbuf[vrow, pl.ds(r16 * 16, 16)]
        for k in range(16):
            v = vv[k]
            r = r16 * 16 + k
            for j in range(8):
                sl = pl.ds(j * 16, 16)
                rows_ref[r, sl] = rows_ref[r, sl] * v


def _sc_body(xh, xc, xs, rh, ch, vh, rc, cc, vc, rs, cs, vs, out,
             acc, rowsA, rowsB, rbuf, cbuf, vbuf, zbuf,
             semA, semB, semI, semZ, semSA, semSB, semR, semV):
    c = lax.axis_index("c")
    s = lax.axis_index("s")
    w = c * 16 + s
    start_w = _CPW * w + jnp.minimum(w, _CREM)
    rbase = s * _RPS

    z16 = jnp.zeros((16,), jnp.float32)

    def zb_body(r, carry):
        for j in range(8):
            zbuf[r, pl.ds(j * 16, 16)] = z16
        return carry

    lax.fori_loop(0, 64, zb_body, None)

    for l, (x_ref, r_ref, c_ref, v_ref) in enumerate(
            ((xh, rh, ch, vh), (xc, rc, cc, vc), (xs, rs, cs, vs))):
        @pl.when(s < 15)
        def _zero_full():
            for k in range(_RPS // 64):
                pltpu.async_copy(zbuf, acc.at[pl.ds(rbase + k * 64, 64)], semZ)
            for k in range(_RPS // 64):
                pltpu.make_async_copy(
                    zbuf, acc.at[pl.ds(rbase + k * 64, 64)], semZ).wait()

        @pl.when(s == 15)
        def _zero_last():
            for k in range(_RPS_LAST // 64):
                pltpu.async_copy(zbuf, acc.at[pl.ds(rbase + k * 64, 64)], semZ)
            rem = _RPS_LAST - (_RPS_LAST // 64) * 64
            if rem:
                pltpu.async_copy(zbuf.at[pl.ds(0, rem)],
                                 acc.at[pl.ds(rbase + _RPS_LAST - rem, rem)],
                                 semZ)
            for k in range(_RPS_LAST // 64):
                pltpu.make_async_copy(
                    zbuf, acc.at[pl.ds(rbase + k * 64, 64)], semZ).wait()
            if rem:
                pltpu.make_async_copy(
                    zbuf.at[pl.ds(0, rem)],
                    acc.at[pl.ds(rbase + _RPS_LAST - rem, rem)], semZ).wait()

        plsc.subcore_barrier()

        def block(b, carry):
            base = start_w + _BPC * b
            a0 = pl.multiple_of((base // 8) * 8, 8)
            o = base - a0
            pltpu.async_copy(c_ref.at[pl.ds(a0, _BLD)], cbuf, semI)
            pltpu.async_copy(r_ref.at[pl.ds(a0, _BLD)], rbuf, semR)
            pltpu.async_copy(v_ref.at[pl.ds(a0, _BLD)], vbuf, semV)
            pltpu.make_async_copy(c_ref.at[pl.ds(a0, _BLD)], cbuf, semI).wait()

            pltpu.async_copy(x_ref.at[cbuf.at[o]], rowsA, semA)
            pltpu.make_async_copy(r_ref.at[pl.ds(a0, _BLD)], rbuf, semR).wait()
            pltpu.make_async_copy(v_ref.at[pl.ds(a0, _BLD)], vbuf, semV).wait()

            def chunk(j, carry2):
                even = (j % 2) == 0

                @pl.when(even)
                def _procA():
                    pltpu.make_async_copy(x_ref.at[pl.ds(0, _CH)], rowsA,
                                          semA).wait()

                    @pl.when(j >= 2)
                    def _wsB():
                        pltpu.make_async_copy(rowsB, acc.at[rbuf.at[o]],
                                              semSB).wait()

                    @pl.when(j < _BPC - 1)
                    def _pfB():
                        pltpu.async_copy(x_ref.at[cbuf.at[o + j + 1]], rowsB,
                                         semB)

                    _mul_rows(rowsA, vbuf, o + j, _CH)
                    pltpu.async_copy(rowsA, acc.at[rbuf.at[o + j]], semSA,
                                     add=True)

                @pl.when(~even)
                def _procB():
                    pltpu.make_async_copy(x_ref.at[pl.ds(0, _CH)], rowsB,
                                          semB).wait()

                    pltpu.make_async_copy(rowsA, acc.at[rbuf.at[o]],
                                          semSA).wait()

                    @pl.when(j < _BPC - 1)
                    def _pfA():
                        pltpu.async_copy(x_ref.at[cbuf.at[o + j + 1]], rowsA,
                                         semA)

                    _mul_rows(rowsB, vbuf, o + j, _CH)
                    pltpu.async_copy(rowsB, acc.at[rbuf.at[o + j]], semSB,
                                     add=True)

                return carry2

            lax.fori_loop(0, _BPC, chunk, None)
            pltpu.make_async_copy(rowsA, acc.at[rbuf.at[o]], semSA).wait()
            return carry

        lax.fori_loop(0, _CPW // _BPC, block, None)

        @pl.when(w < _CREM)
        def _extra():
            base = start_w + _CPW
            a0 = pl.multiple_of((base // 8) * 8, 8)
            o = base - a0
            pltpu.sync_copy(r_ref.at[pl.ds(a0, 8)], rbuf.at[pl.ds(0, 8)])
            pltpu.sync_copy(c_ref.at[pl.ds(a0, 8)], cbuf.at[pl.ds(0, 8)])
            pltpu.sync_copy(v_ref.at[pl.ds(a0, 8)], vbuf.at[pl.ds(0, 8)])
            pltpu.async_copy(x_ref.at[cbuf.at[o]], rowsA, semA).wait()
            _mul_rows(rowsA, vbuf, o, _CH)
            pltpu.sync_copy(rowsA, acc.at[rbuf.at[o]], add=True)

        plsc.subcore_barrier()

        @pl.when(s < 15)
        def _flush_full():
            pltpu.sync_copy(acc.at[pl.ds(rbase, _RPS)],
                            out.at[l, c, pl.ds(rbase, _RPS)])

        @pl.when(s == 15)
        def _flush_last():
            pltpu.sync_copy(acc.at[pl.ds(rbase, _RPS_LAST)],
                            out.at[l, c, pl.ds(rbase, _RPS_LAST)])


def _sc_spmm(xh, xc, xs, rh, ch, vh, rc, cc, vc, rs, cs, vs):
    mesh = plsc.VectorSubcoreMesh(core_axis_name="c", subcore_axis_name="s")
    f = pl.kernel(
        _sc_body,
        mesh=mesh,
        out_type=jax.ShapeDtypeStruct((3, 2, _N, _C), jnp.float32),
        scratch_types=[
            pltpu.VMEM_SHARED((_N, _C), jnp.float32),
            pltpu.VMEM((_CH, _C), jnp.float32),
            pltpu.VMEM((_CH, _C), jnp.float32),
            pltpu.VMEM((_BLD, _CH), jnp.int32),
            pltpu.VMEM((_BLD, _CH), jnp.int32),
            pltpu.VMEM((_BLD, _CH), jnp.float32),
            pltpu.VMEM((64, _C), jnp.float32),
            pltpu.SemaphoreType.DMA,
            pltpu.SemaphoreType.DMA,
            pltpu.SemaphoreType.DMA,
            pltpu.SemaphoreType.DMA,
            pltpu.SemaphoreType.DMA,
            pltpu.SemaphoreType.DMA,
            pltpu.SemaphoreType.DMA,
            pltpu.SemaphoreType.DMA,
        ],
    )
    return f(xh, xc, xs, rh, ch, vh, rc, cc, vc, rs, cs, vs)


def _chunk_grid(idx, val):
    rows = jnp.pad(idx[0].astype(jnp.int32).reshape(_NCHUNK, _CH),
                   ((0, _NPAD - _NCHUNK), (0, 0)))
    cols = jnp.pad(idx[1].astype(jnp.int32).reshape(_NCHUNK, _CH),
                   ((0, _NPAD - _NCHUNK), (0, 0)))
    vals = jnp.pad(val.reshape(_NCHUNK, _CH), ((0, _NPAD - _NCHUNK), (0, 0)))
    return rows, cols, vals


def kernel(X_HypGNet, X_CGNet, X_SGNet, L_hyp_idx, L_hyp_val, L_cg_idx,
           L_cg_val, L_sg_idx, L_sg_val, W_hyp, b_hyp, W_cg, b_cg, W_sg, b_sg):
    w3 = jnp.stack([W_hyp, W_cg, W_sg])
    b3 = jnp.stack([b_hyp, b_cg, b_sg])
    xh, xc, xs = _project(X_HypGNet, X_CGNet, X_SGNet, w3, b3)
    rh, ch, vh = _chunk_grid(L_hyp_idx, L_hyp_val)
    rc, cc, vc = _chunk_grid(L_cg_idx, L_cg_val)
    rs, cs, vs = _chunk_grid(L_sg_idx, L_sg_val)
    part = _sc_spmm(xh, xc, xs, rh, ch, vh, rc, cc, vc, rs, cs, vs)
    oh, oc, os_ = _sum_partials(part)
    return (oh, oc, os_)
